# baseline (speedup 1.0000x reference)
import functools

import jax
import jax.numpy as jnp
from jax import lax
from jax.experimental import pallas as pl
from jax.experimental.pallas import tpu as pltpu
from jax.experimental.pallas import tpu_sc as plsc

B = 16384
EMB = 32
EPS = 1e-5

NC = 2
NS = 16
NW = NC * NS
BPW = B // NW
RB = 32
NR = BPW // RB
LB = 1024


def _pack_body(t_ref, out_ref):
    blkT = jnp.transpose(t_ref[...])
    x3 = blkT.reshape(LB // 4, 4, EMB)
    out_ref[...] = jnp.concatenate([x3[:, q, :] for q in range(4)], axis=1)


def _pack(tT, n):
    grid = (n + LB - 1) // LB
    return pl.pallas_call(
        _pack_body,
        grid=(grid,),
        in_specs=[pl.BlockSpec((EMB, LB), lambda i: (0, i))],
        out_specs=pl.BlockSpec((LB // 4, 128), lambda i: (i, 0)),
        out_shape=jax.ShapeDtypeStruct((n // 4, 128), jnp.float32),
    )(tT)


def _sc_gather_body(uid_ref, mid_ref, t_umf, t_mmf, t_umlp, t_mmlp,
                    o_umf, o_mmf, o_umlp, o_mmlp,
                    idx_u, idx_m, r_umf, r_mmf, r_umlp, r_mmlp, sem):
    wid = lax.axis_index("s") * NC + lax.axis_index("c")
    base = wid * BPW
    pltpu.sync_copy(uid_ref.at[wid], idx_u)
    pltpu.sync_copy(mid_ref.at[wid], idx_m)

    for r in range(NR):
        iu = idx_u.at[r]
        im = idx_m.at[r]
        cs = [pltpu.async_copy(t_umf.at[iu], r_umf, sem),
              pltpu.async_copy(t_mmf.at[im], r_mmf, sem),
              pltpu.async_copy(t_umlp.at[iu], r_umlp, sem),
              pltpu.async_copy(t_mmlp.at[im], r_mmlp, sem)]
        for c in cs:
            c.wait()
        out_slc = pl.ds(base + r * RB, RB)
        pltpu.sync_copy(r_umf, o_umf.at[out_slc])
        pltpu.sync_copy(r_mmf, o_mmf.at[out_slc])
        pltpu.sync_copy(r_umlp, o_umlp.at[out_slc])
        pltpu.sync_copy(r_mmlp, o_mmlp.at[out_slc])


@functools.partial(jax.jit, static_argnums=())
def _sc_gather(uidp, midp, t_umf, t_mmf, t_umlp, t_mmlp):
    mesh = plsc.VectorSubcoreMesh(core_axis_name="c", subcore_axis_name="s")
    rows = jax.ShapeDtypeStruct((B, 128), jnp.float32)
    return pl.kernel(
        _sc_gather_body,
        out_type=(rows, rows, rows, rows),
        mesh=mesh,
        scratch_types=(
            pltpu.VMEM((NR, RB), jnp.int32),
            pltpu.VMEM((NR, RB), jnp.int32),
            pltpu.VMEM((RB, 128), jnp.float32),
            pltpu.VMEM((RB, 128), jnp.float32),
            pltpu.VMEM((RB, 128), jnp.float32),
            pltpu.VMEM((RB, 128), jnp.float32),
            pltpu.SemaphoreType.DMA,
        ),
    )(uidp, midp, t_umf, t_mmf, t_umlp, t_mmlp)


CH = 512
NCH = B // CH


def _dense_body(u_mf4, m_mf4, u_mlp4, m_mlp4, urem, mrem,
                W0, b0, g0, be0, W1, b1, g1, be1, W2, b2, g2, be2, Wo, bo,
                out_ref, h0, h1, h2, mf_buf):
    f32 = jnp.float32
    hi = jax.lax.Precision.HIGHEST

    def dgen(W, x):
        return lax.dot_general(W, x, ((((1,), (0,))), ((), ())),
                               preferred_element_type=f32, precision=hi)

    W0v = W0[...]
    W0l, W0r = W0v[:, :EMB], W0v[:, EMB:]
    b0v, b1v, b2v = b0[...], b1[...], b2[...]

    def sel32(x4ref, sl, rem):
        y = jnp.transpose(x4ref[sl, :])
        acc = None
        for q in range(4):
            m = jnp.where(rem == float(q), 1.0, 0.0)
            part = y[q * EMB:(q + 1) * EMB, :] * m
            acc = part if acc is None else acc + part
        return acc

    for c in range(NCH):
        sl = pl.ds(c * CH, CH)
        ur = urem[0:1, sl]
        mr = mrem[0:1, sl]
        um = sel32(u_mlp4, sl, ur)
        mm = sel32(m_mlp4, sl, mr)
        h = dgen(W0l, um) + dgen(W0r, mm)
        h0[:, sl] = jnp.maximum(h + b0v[:, None], 0.0)
        uf = sel32(u_mf4, sl, ur)
        mf_ = sel32(m_mf4, sl, mr)
        mf_buf[0:1, sl] = jnp.sum(uf * mf_, axis=0).reshape(1, CH)

    def stats(href, g, be):
        hv = href[...]
        mean = jnp.sum(hv, axis=1) * (1.0 / B)
        var = jnp.sum(hv * hv, axis=1) * (1.0 / B) - mean * mean
        v = var + EPS
        r = lax.rsqrt(v)
        r = r * (1.5 - 0.5 * v * r * r)
        scale = g[...] * r
        shift = be[...] - mean * scale
        return scale, shift

    scale0, shift0 = stats(h0, g0, be0)

    for c in range(NCH):
        sl = pl.ds(c * CH, CH)
        x = h0[:, sl] * scale0[:, None] + shift0[:, None]
        h1[:, sl] = jnp.maximum(dgen(W1[...], x) + b1v[:, None], 0.0)

    scale1, shift1 = stats(h1, g1, be1)

    for c in range(NCH):
        sl = pl.ds(c * CH, CH)
        x = h1[:, sl] * scale1[:, None] + shift1[:, None]
        h2[:, sl] = jnp.maximum(dgen(W2[...], x) + b2v[:, None], 0.0)

    scale2, shift2 = stats(h2, g2, be2)

    Wov = Wo[...]
    w_mlp = Wov[:, 1:].reshape(16, 1)
    w_mf = Wov[:, 0:1]
    bov = bo[...]
    for c in range(NCH):
        sl = pl.ds(c * CH, CH)
        x = h2[:, sl] * scale2[:, None] + shift2[:, None]
        acc = jnp.sum(x * w_mlp, axis=0).reshape(1, CH)
        out_ref[0:1, sl] = mf_buf[0:1, sl] * w_mf + acc + bov[None, :]


def _dense(u_mf4, m_mf4, u_mlp4, m_mlp4, urem, mrem, weights):
    return pl.pallas_call(
        _dense_body,
        out_shape=jax.ShapeDtypeStruct((1, B), jnp.float32),
        scratch_shapes=[
            pltpu.VMEM((64, B), jnp.float32),
            pltpu.VMEM((32, B), jnp.float32),
            pltpu.VMEM((16, B), jnp.float32),
            pltpu.VMEM((1, B), jnp.float32),
        ],
    )(u_mf4, m_mf4, u_mlp4, m_mlp4, urem, mrem, *weights)


def kernel(user_ids, movie_ids, ue_mf, me_mf, ue_mlp, me_mlp,
           W0, b0, g0, be0, W1, b1, g1, be1, W2, b2, g2, be2, Wo, bo):
    uid = user_ids.astype(jnp.int32)
    mid = movie_ids.astype(jnp.int32)
    uidp = (uid // 4).reshape(NW, NR, RB)
    midp = (mid // 4).reshape(NW, NR, RB)
    urem = (uid % 4).astype(jnp.float32).reshape(1, B)
    mrem = (mid % 4).astype(jnp.float32).reshape(1, B)

    tu_mf = _pack(ue_mf.T, ue_mf.shape[0])
    tm_mf = _pack(me_mf.T, me_mf.shape[0])
    tu_mlp = _pack(ue_mlp.T, ue_mlp.shape[0])
    tm_mlp = _pack(me_mlp.T, me_mlp.shape[0])

    u_mf4, m_mf4, u_mlp4, m_mlp4 = _sc_gather(
        uidp, midp, tu_mf, tm_mf, tu_mlp, tm_mlp)
    weights = (W0, b0, g0, be0, W1, b1, g1, be1, W2, b2, g2, be2, Wo, bo)
    out = _dense(u_mf4, m_mf4, u_mlp4, m_mlp4, urem, mrem, weights)
    return out.reshape(B)


# speedup vs baseline: 2.2539x; 2.2539x over previous
import functools

import jax
import jax.numpy as jnp
from jax import lax
from jax.experimental import pallas as pl
from jax.experimental.pallas import tpu as pltpu
from jax.experimental.pallas import tpu_sc as plsc

B = 16384
EMB = 32
EPS = 1e-5

NC = 2
NS = 16
NW = NC * NS
BPW = B // NW
RB = 32
NR = BPW // RB
LB = 4096
QL = LB // 4


def _pack_body(t_ref, out_ref):
    for q in range(4):
        out_ref[:, q * EMB:(q + 1) * EMB] = jnp.transpose(
            t_ref[:, pl.ds(q * QL, QL)])


def _pack(tT, n):
    grid = (n + LB - 1) // LB
    return pl.pallas_call(
        _pack_body,
        grid=(grid,),
        in_specs=[pl.BlockSpec((EMB, LB), lambda i: (0, i))],
        out_specs=pl.BlockSpec((QL, 128), lambda i: (i, 0)),
        out_shape=jax.ShapeDtypeStruct((grid * QL, 128), jnp.float32),
    )(tT)


def _sc_gather_body(uid_ref, mid_ref, t_umf, t_mmf, t_umlp, t_mmlp,
                    o_umf, o_mmf, o_umlp, o_mmlp,
                    idx_u, idx_m, r_umf, r_mmf, r_umlp, r_mmlp, sem):
    wid = lax.axis_index("s") * NC + lax.axis_index("c")
    base = wid * BPW
    pltpu.sync_copy(uid_ref.at[wid], idx_u)
    pltpu.sync_copy(mid_ref.at[wid], idx_m)

    for r in range(NR):
        iu = idx_u.at[r]
        im = idx_m.at[r]
        cs = [pltpu.async_copy(t_umf.at[iu], r_umf, sem),
              pltpu.async_copy(t_mmf.at[im], r_mmf, sem),
              pltpu.async_copy(t_umlp.at[iu], r_umlp, sem),
              pltpu.async_copy(t_mmlp.at[im], r_mmlp, sem)]
        for c in cs:
            c.wait()
        out_slc = pl.ds(base + r * RB, RB)
        pltpu.sync_copy(r_umf, o_umf.at[out_slc])
        pltpu.sync_copy(r_mmf, o_mmf.at[out_slc])
        pltpu.sync_copy(r_umlp, o_umlp.at[out_slc])
        pltpu.sync_copy(r_mmlp, o_mmlp.at[out_slc])


@functools.partial(jax.jit, static_argnums=())
def _sc_gather(uidp, midp, t_umf, t_mmf, t_umlp, t_mmlp):
    mesh = plsc.VectorSubcoreMesh(core_axis_name="c", subcore_axis_name="s")
    rows = jax.ShapeDtypeStruct((B, 128), jnp.float32)
    return pl.kernel(
        _sc_gather_body,
        out_type=(rows, rows, rows, rows),
        mesh=mesh,
        scratch_types=(
            pltpu.VMEM((NR, RB), jnp.int32),
            pltpu.VMEM((NR, RB), jnp.int32),
            pltpu.VMEM((RB, 128), jnp.float32),
            pltpu.VMEM((RB, 128), jnp.float32),
            pltpu.VMEM((RB, 128), jnp.float32),
            pltpu.VMEM((RB, 128), jnp.float32),
            pltpu.SemaphoreType.DMA,
        ),
    )(uidp, midp, t_umf, t_mmf, t_umlp, t_mmlp)


CH = 512
NCH = B // CH


def _dense_body(u_mf4, m_mf4, u_mlp4, m_mlp4, urem, mrem,
                W0, b0, g0, be0, W1, b1, g1, be1, W2, b2, g2, be2, Wo, bo,
                out_ref, h0, h1, h2, mf_buf):
    f32 = jnp.float32
    hi = jax.lax.Precision.HIGHEST

    def dgen(W, x):
        return lax.dot_general(W, x, ((((1,), (0,))), ((), ())),
                               preferred_element_type=f32, precision=hi)

    W0v = W0[...]
    W0l, W0r = W0v[:, :EMB], W0v[:, EMB:]
    b0v, b1v, b2v = b0[...], b1[...], b2[...]

    def sel32(x4ref, sl, rem):
        y = jnp.transpose(x4ref[sl, :])
        acc = None
        for q in range(4):
            part = jnp.where(rem == float(q), y[q * EMB:(q + 1) * EMB, :], 0.0)
            acc = part if acc is None else acc + part
        return acc

    for c in range(NCH):
        sl = pl.ds(c * CH, CH)
        ur = urem[0:1, sl]
        mr = mrem[0:1, sl]
        um = sel32(u_mlp4, sl, ur)
        mm = sel32(m_mlp4, sl, mr)
        h = dgen(W0l, um) + dgen(W0r, mm)
        h0[:, sl] = jnp.maximum(h + b0v[:, None], 0.0)
        uf = sel32(u_mf4, sl, ur)
        mf_ = sel32(m_mf4, sl, mr)
        mf_buf[0:1, sl] = jnp.sum(uf * mf_, axis=0).reshape(1, CH)

    def stats(href, g, be):
        hv = href[...]
        mean = jnp.sum(hv, axis=1) * (1.0 / B)
        var = jnp.sum(hv * hv, axis=1) * (1.0 / B) - mean * mean
        v = var + EPS
        r = lax.rsqrt(v)
        r = r * (1.5 - 0.5 * v * r * r)
        scale = g[...] * r
        shift = be[...] - mean * scale
        return scale, shift

    scale0, shift0 = stats(h0, g0, be0)

    for c in range(NCH):
        sl = pl.ds(c * CH, CH)
        x = h0[:, sl] * scale0[:, None] + shift0[:, None]
        h1[:, sl] = jnp.maximum(dgen(W1[...], x) + b1v[:, None], 0.0)

    scale1, shift1 = stats(h1, g1, be1)

    for c in range(NCH):
        sl = pl.ds(c * CH, CH)
        x = h1[:, sl] * scale1[:, None] + shift1[:, None]
        h2[:, sl] = jnp.maximum(dgen(W2[...], x) + b2v[:, None], 0.0)

    scale2, shift2 = stats(h2, g2, be2)

    Wov = Wo[...]
    w_mlp = Wov[:, 1:].reshape(16, 1)
    w_mf = Wov[:, 0:1]
    bov = bo[...]
    for c in range(NCH):
        sl = pl.ds(c * CH, CH)
        x = h2[:, sl] * scale2[:, None] + shift2[:, None]
        acc = jnp.sum(x * w_mlp, axis=0).reshape(1, CH)
        out_ref[0:1, sl] = mf_buf[0:1, sl] * w_mf + acc + bov[None, :]


def _dense(u_mf4, m_mf4, u_mlp4, m_mlp4, urem, mrem, weights):
    return pl.pallas_call(
        _dense_body,
        out_shape=jax.ShapeDtypeStruct((1, B), jnp.float32),
        scratch_shapes=[
            pltpu.VMEM((64, B), jnp.float32),
            pltpu.VMEM((32, B), jnp.float32),
            pltpu.VMEM((16, B), jnp.float32),
            pltpu.VMEM((1, B), jnp.float32),
        ],
    )(u_mf4, m_mf4, u_mlp4, m_mlp4, urem, mrem, *weights)


def kernel(user_ids, movie_ids, ue_mf, me_mf, ue_mlp, me_mlp,
           W0, b0, g0, be0, W1, b1, g1, be1, W2, b2, g2, be2, Wo, bo):
    uid = user_ids.astype(jnp.int32)
    mid = movie_ids.astype(jnp.int32)
    uidp = ((uid // LB) * QL + uid % QL).reshape(NW, NR, RB)
    midp = ((mid // LB) * QL + mid % QL).reshape(NW, NR, RB)
    urem = ((uid // QL) % 4).astype(jnp.float32).reshape(1, B)
    mrem = ((mid // QL) % 4).astype(jnp.float32).reshape(1, B)

    tu_mf = _pack(ue_mf.T, ue_mf.shape[0])
    tm_mf = _pack(me_mf.T, me_mf.shape[0])
    tu_mlp = _pack(ue_mlp.T, ue_mlp.shape[0])
    tm_mlp = _pack(me_mlp.T, me_mlp.shape[0])

    u_mf4, m_mf4, u_mlp4, m_mlp4 = _sc_gather(
        uidp, midp, tu_mf, tm_mf, tu_mlp, tm_mlp)
    weights = (W0, b0, g0, be0, W1, b1, g1, be1, W2, b2, g2, be2, Wo, bo)
    out = _dense(u_mf4, m_mf4, u_mlp4, m_mlp4, urem, mrem, weights)
    return out.reshape(B)


# speedup vs baseline: 4.2074x; 1.8667x over previous
import functools

import jax
import jax.numpy as jnp
from jax import lax
from jax.experimental import pallas as pl
from jax.experimental.pallas import tpu as pltpu
from jax.experimental.pallas import tpu_sc as plsc

B = 16384
EMB = 32
EPS = 1e-5

NC = 2
NS = 16
NW = NC * NS
BPW = B // NW
RB = 32
NR = BPW // RB
LB = 4096
QL = LB // 4


def _pack_body(ta_ref, tb_ref, oa_ref, ob_ref):
    for t_ref, o_ref in ((ta_ref, oa_ref), (tb_ref, ob_ref)):
        x = jnp.concatenate(
            [t_ref[:, pl.ds(q * QL, QL)] for q in range(4)], axis=0)
        o_ref[...] = jnp.transpose(x)


def _pack2(tTa, tTb, n):
    grid = (n + LB - 1) // LB
    spec_in = pl.BlockSpec((EMB, LB), lambda i: (0, i))
    spec_out = pl.BlockSpec((QL, 128), lambda i: (i, 0))
    oshape = jax.ShapeDtypeStruct((grid * QL, 128), jnp.float32)
    return pl.pallas_call(
        _pack_body,
        grid=(grid,),
        in_specs=[spec_in, spec_in],
        out_specs=[spec_out, spec_out],
        out_shape=[oshape, oshape],
    )(tTa, tTb)


def _sc_gather_body(uid_ref, mid_ref, t_umf, t_mmf, t_umlp, t_mmlp,
                    o_umf, o_mmf, o_umlp, o_mmlp,
                    idx_u, idx_m, r_umf, r_mmf, r_umlp, r_mmlp, sem):
    wid = lax.axis_index("s") * NC + lax.axis_index("c")
    base = wid * BPW
    pltpu.sync_copy(uid_ref.at[wid], idx_u)
    pltpu.sync_copy(mid_ref.at[wid], idx_m)

    for r in range(NR):
        iu = idx_u.at[r]
        im = idx_m.at[r]
        cs = [pltpu.async_copy(t_umf.at[iu], r_umf, sem),
              pltpu.async_copy(t_mmf.at[im], r_mmf, sem),
              pltpu.async_copy(t_umlp.at[iu], r_umlp, sem),
              pltpu.async_copy(t_mmlp.at[im], r_mmlp, sem)]
        for c in cs:
            c.wait()
        out_slc = pl.ds(base + r * RB, RB)
        pltpu.sync_copy(r_umf, o_umf.at[out_slc])
        pltpu.sync_copy(r_mmf, o_mmf.at[out_slc])
        pltpu.sync_copy(r_umlp, o_umlp.at[out_slc])
        pltpu.sync_copy(r_mmlp, o_mmlp.at[out_slc])


@functools.partial(jax.jit, static_argnums=())
def _sc_gather(uidp, midp, t_umf, t_mmf, t_umlp, t_mmlp):
    mesh = plsc.VectorSubcoreMesh(core_axis_name="c", subcore_axis_name="s")
    rows = jax.ShapeDtypeStruct((B, 128), jnp.float32)
    return pl.kernel(
        _sc_gather_body,
        out_type=(rows, rows, rows, rows),
        mesh=mesh,
        scratch_types=(
            pltpu.VMEM((NR, RB), jnp.int32),
            pltpu.VMEM((NR, RB), jnp.int32),
            pltpu.VMEM((RB, 128), jnp.float32),
            pltpu.VMEM((RB, 128), jnp.float32),
            pltpu.VMEM((RB, 128), jnp.float32),
            pltpu.VMEM((RB, 128), jnp.float32),
            pltpu.SemaphoreType.DMA,
        ),
    )(uidp, midp, t_umf, t_mmf, t_umlp, t_mmlp)


CH = 512
NCH = B // CH


def _dense_body(u_mf4, m_mf4, u_mlp4, m_mlp4, urem, mrem,
                W0, b0, g0, be0, W1, b1, g1, be1, W2, b2, g2, be2, Wo, bo,
                out_ref, h0, h1, h2, mf_buf):
    f32 = jnp.float32
    hi = jax.lax.Precision.HIGHEST

    def dgen(W, x):
        return lax.dot_general(W, x, ((((1,), (0,))), ((), ())),
                               preferred_element_type=f32, precision=hi)

    W0v = W0[...]
    W0l, W0r = W0v[:, :EMB], W0v[:, EMB:]
    b0v, b1v, b2v = b0[...], b1[...], b2[...]

    def sel32(x4ref, sl, rem):
        y = jnp.transpose(x4ref[sl, :])
        acc = None
        for q in range(4):
            part = jnp.where(rem == float(q), y[q * EMB:(q + 1) * EMB, :], 0.0)
            acc = part if acc is None else acc + part
        return acc

    for c in range(NCH):
        sl = pl.ds(c * CH, CH)
        ur = urem[0:1, sl]
        mr = mrem[0:1, sl]
        um = sel32(u_mlp4, sl, ur)
        mm = sel32(m_mlp4, sl, mr)
        h = dgen(W0l, um) + dgen(W0r, mm)
        h0[:, sl] = jnp.maximum(h + b0v[:, None], 0.0)
        uf = sel32(u_mf4, sl, ur)
        mf_ = sel32(m_mf4, sl, mr)
        mf_buf[0:1, sl] = jnp.sum(uf * mf_, axis=0).reshape(1, CH)

    def stats(href, g, be):
        hv = href[...]
        mean = jnp.sum(hv, axis=1) * (1.0 / B)
        var = jnp.sum(hv * hv, axis=1) * (1.0 / B) - mean * mean
        v = var + EPS
        r = lax.rsqrt(v)
        r = r * (1.5 - 0.5 * v * r * r)
        scale = g[...] * r
        shift = be[...] - mean * scale
        return scale, shift

    scale0, shift0 = stats(h0, g0, be0)

    for c in range(NCH):
        sl = pl.ds(c * CH, CH)
        x = h0[:, sl] * scale0[:, None] + shift0[:, None]
        h1[:, sl] = jnp.maximum(dgen(W1[...], x) + b1v[:, None], 0.0)

    scale1, shift1 = stats(h1, g1, be1)

    for c in range(NCH):
        sl = pl.ds(c * CH, CH)
        x = h1[:, sl] * scale1[:, None] + shift1[:, None]
        h2[:, sl] = jnp.maximum(dgen(W2[...], x) + b2v[:, None], 0.0)

    scale2, shift2 = stats(h2, g2, be2)

    Wov = Wo[...]
    w_mlp = Wov[:, 1:].reshape(16, 1)
    w_mf = Wov[:, 0:1]
    bov = bo[...]
    for c in range(NCH):
        sl = pl.ds(c * CH, CH)
        x = h2[:, sl] * scale2[:, None] + shift2[:, None]
        acc = jnp.sum(x * w_mlp, axis=0).reshape(1, CH)
        out_ref[0:1, sl] = mf_buf[0:1, sl] * w_mf + acc + bov[None, :]


def _dense(u_mf4, m_mf4, u_mlp4, m_mlp4, urem, mrem, weights):
    return pl.pallas_call(
        _dense_body,
        out_shape=jax.ShapeDtypeStruct((1, B), jnp.float32),
        scratch_shapes=[
            pltpu.VMEM((64, B), jnp.float32),
            pltpu.VMEM((32, B), jnp.float32),
            pltpu.VMEM((16, B), jnp.float32),
            pltpu.VMEM((1, B), jnp.float32),
        ],
    )(u_mf4, m_mf4, u_mlp4, m_mlp4, urem, mrem, *weights)


def kernel(user_ids, movie_ids, ue_mf, me_mf, ue_mlp, me_mlp,
           W0, b0, g0, be0, W1, b1, g1, be1, W2, b2, g2, be2, Wo, bo):
    uid = user_ids.astype(jnp.int32)
    mid = movie_ids.astype(jnp.int32)
    uidp = ((uid // LB) * QL + uid % QL).reshape(NW, NR, RB)
    midp = ((mid // LB) * QL + mid % QL).reshape(NW, NR, RB)
    urem = ((uid // QL) % 4).astype(jnp.float32).reshape(1, B)
    mrem = ((mid // QL) % 4).astype(jnp.float32).reshape(1, B)

    tu_mf, tu_mlp = _pack2(ue_mf.T, ue_mlp.T, ue_mf.shape[0])
    tm_mf, tm_mlp = _pack2(me_mf.T, me_mlp.T, me_mf.shape[0])

    u_mf4, m_mf4, u_mlp4, m_mlp4 = _sc_gather(
        uidp, midp, tu_mf, tm_mf, tu_mlp, tm_mlp)
    weights = (W0, b0, g0, be0, W1, b1, g1, be1, W2, b2, g2, be2, Wo, bo)
    out = _dense(u_mf4, m_mf4, u_mlp4, m_mlp4, urem, mrem, weights)
    return out.reshape(B)
